# Initial kernel scaffold; baseline (speedup 1.0000x reference)
import jax
import jax.numpy as jnp
from jax.experimental import pallas as pl


def kernel(char_ids, table):
    raise NotImplementedError("write your pallas kernel here")


# speedup vs baseline: 3.9709x; 3.9709x over previous
import functools

import jax
import jax.numpy as jnp
from jax import lax
from jax.experimental import pallas as pl
from jax.experimental.pallas import tpu as pltpu
from jax.experimental.pallas import tpu_sc as plsc

B = 4096
L = 50
D = 64
LANES = 16
NC = 2
NS = 16
NW = NC * NS
ROWS_PER_W = B // NW
CHUNK_ROWS = 2
IDX_RAW = CHUNK_ROWS * L
IDX_PAD = 104
CHUNKS_PER_W = ROWS_PER_W // CHUNK_ROWS
NBUF = 2


def _worker_body(idx_hbm, table_hbm, out_hbm, idx_v, rows0, rows1, out_v,
                 sem0, sem1):
  wid = lax.axis_index("s") * NC + lax.axis_index("c")
  base_chunk = wid * CHUNKS_PER_W
  pltpu.sync_copy(idx_hbm.at[pl.ds(base_chunk, CHUNKS_PER_W)], idx_v)
  rows = (rows0, rows1)
  sems = (sem0, sem1)

  def gather(j, b):
    return pltpu.make_async_copy(table_hbm.at[idx_v.at[j]], rows[b], sems[b])

  for b in range(NBUF):
    gather(b, b).start()

  def step(p, carry):
    for b in range(NBUF):
      j = p * NBUF + b
      gather(j, b).wait()
      nxt = j + NBUF

      @pl.when(nxt < CHUNKS_PER_W)
      def _():
        gather(nxt, b).start()

      buf = rows[b]
      for r in range(CHUNK_ROWS):
        base = r * L
        for d in range(D // LANES):
          acc = buf[base, pl.ds(d * LANES, LANES)]
          for s in range(1, L):
            acc = jnp.maximum(acc, buf[base + s, pl.ds(d * LANES, LANES)])
          out_v[j * CHUNK_ROWS + r, pl.ds(d * LANES, LANES)] = acc
    return carry

  lax.fori_loop(0, CHUNKS_PER_W // NBUF, step, None)
  pltpu.sync_copy(out_v, out_hbm.at[pl.ds(wid * ROWS_PER_W, ROWS_PER_W)])


@functools.partial(
    pl.kernel,
    out_type=jax.ShapeDtypeStruct((B, D), jnp.float32),
    mesh=plsc.VectorSubcoreMesh(core_axis_name="c", subcore_axis_name="s"),
    scratch_types=[
        pltpu.VMEM((CHUNKS_PER_W, IDX_PAD), jnp.int32),
        pltpu.VMEM((IDX_PAD, D), jnp.float32),
        pltpu.VMEM((IDX_PAD, D), jnp.float32),
        pltpu.VMEM((ROWS_PER_W, D), jnp.float32),
        pltpu.SemaphoreType.DMA,
        pltpu.SemaphoreType.DMA,
    ],
    compiler_params=pltpu.CompilerParams(use_tc_tiling_on_sc=False),
)
def _sc_embed_maxpool(idx_hbm, table_hbm, out_hbm, idx_v, rows0, rows1,
                      out_v, sem0, sem1):
  _worker_body(idx_hbm, table_hbm, out_hbm, idx_v, rows0, rows1, out_v,
               sem0, sem1)


def kernel(char_ids, table):
  idx = char_ids.astype(jnp.int32).reshape(NW * CHUNKS_PER_W, IDX_RAW)
  idx = jnp.pad(idx, ((0, 0), (0, IDX_PAD - IDX_RAW)))
  return _sc_embed_maxpool(idx, table)


# speedup vs baseline: 10.0634x; 2.5343x over previous
import functools

import jax
import jax.numpy as jnp
from jax import lax
from jax.experimental import pallas as pl
from jax.experimental.pallas import tpu as pltpu
from jax.experimental.pallas import tpu_sc as plsc

B = 4096
L = 50
D = 64
LANES = 16
NC = 2
NS = 16
NW = NC * NS
ROWS_PER_W = B // NW
CHUNK_ROWS = 8
IDX_RAW = CHUNK_ROWS * L
IDX_PAD = 400
CHUNKS_PER_W = ROWS_PER_W // CHUNK_ROWS
NBUF = 2


def _worker_body(idx_hbm, table_hbm, out_hbm, idx_v, rows0, rows1, out_v,
                 sem0, sem1):
  wid = lax.axis_index("s") * NC + lax.axis_index("c")
  base_chunk = wid * CHUNKS_PER_W
  pltpu.sync_copy(idx_hbm.at[pl.ds(base_chunk, CHUNKS_PER_W)], idx_v)
  rows = (rows0, rows1)
  sems = (sem0, sem1)

  def gather(j, b):
    return pltpu.make_async_copy(table_hbm.at[idx_v.at[j]], rows[b], sems[b])

  for b in range(NBUF):
    gather(b, b).start()

  def step(p, carry):
    for b in range(NBUF):
      j = p * NBUF + b
      gather(j, b).wait()
      nxt = j + NBUF

      @pl.when(nxt < CHUNKS_PER_W)
      def _():
        gather(nxt, b).start()

      buf = rows[b]

      def row_body(r, carry2):
        base = r * L
        for d in range(D // LANES):
          acc = buf[base, pl.ds(d * LANES, LANES)]
          for s in range(1, L):
            acc = jnp.maximum(acc, buf[base + s, pl.ds(d * LANES, LANES)])
          out_v[j * CHUNK_ROWS + r, pl.ds(d * LANES, LANES)] = acc
        return carry2

      lax.fori_loop(0, CHUNK_ROWS, row_body, None)
    return carry

  lax.fori_loop(0, CHUNKS_PER_W // NBUF, step, None)
  pltpu.sync_copy(out_v, out_hbm.at[pl.ds(wid * ROWS_PER_W, ROWS_PER_W)])


@functools.partial(
    pl.kernel,
    out_type=jax.ShapeDtypeStruct((B, D), jnp.float32),
    mesh=plsc.VectorSubcoreMesh(core_axis_name="c", subcore_axis_name="s"),
    scratch_types=[
        pltpu.VMEM((CHUNKS_PER_W, IDX_PAD), jnp.int32),
        pltpu.VMEM((IDX_PAD, D), jnp.float32),
        pltpu.VMEM((IDX_PAD, D), jnp.float32),
        pltpu.VMEM((ROWS_PER_W, D), jnp.float32),
        pltpu.SemaphoreType.DMA,
        pltpu.SemaphoreType.DMA,
    ],
    compiler_params=pltpu.CompilerParams(use_tc_tiling_on_sc=False),
)
def _sc_embed_maxpool(idx_hbm, table_hbm, out_hbm, idx_v, rows0, rows1,
                      out_v, sem0, sem1):
  _worker_body(idx_hbm, table_hbm, out_hbm, idx_v, rows0, rows1, out_v,
               sem0, sem1)


def kernel(char_ids, table):
  idx = char_ids.astype(jnp.int32).reshape(NW * CHUNKS_PER_W, IDX_RAW)
  if IDX_PAD > IDX_RAW:
    idx = jnp.pad(idx, ((0, 0), (0, IDX_PAD - IDX_RAW)))
  return _sc_embed_maxpool(idx, table)
